# Initial kernel scaffold; baseline (speedup 1.0000x reference)
import functools

import jax
import jax.numpy as jnp
from jax import lax
from jax.experimental import pallas as pl
from jax.experimental.pallas import tpu as pltpu
from jax.experimental.pallas import tpu_sc as plsc

N_NODES = 1000000
NBX = NBY = 1024
MAP_WORDS = NBX * NBY

NUM_CORES = 2
NUM_SUBCORES = 16
NW = NUM_CORES * NUM_SUBCORES
LANES = 16

B = 1664
NBLK = 19
C = B * NBLK
NP = NW * C
NR = 9 * B // 128


def _body(xs, ys, sxs, sys_, pws, map_hbm, out_hbm,
          map_sp, xb, yb, sxb, syb, pwb, idxb, wb, valb, outb, sem):
    cid = lax.axis_index("c")
    sid = lax.axis_index("s")
    wid = sid * NUM_CORES + cid

    @pl.when(sid == 0)
    def _():
        pltpu.sync_copy(map_hbm, map_sp)

    plsc.subcore_barrier()

    def block(blk, _):
        base = wid * C + blk * B
        pltpu.sync_copy(xs.at[pl.ds(base, B)], xb)
        pltpu.sync_copy(ys.at[pl.ds(base, B)], yb)
        pltpu.sync_copy(sxs.at[pl.ds(base, B)], sxb)
        pltpu.sync_copy(sys_.at[pl.ds(base, B)], syb)
        pltpu.sync_copy(pws.at[pl.ds(base, B)], pwb)

        def gen(c, _):
            x = xb[pl.ds(c * LANES, LANES)]
            y = yb[pl.ds(c * LANES, LANES)]
            sx = sxb[pl.ds(c * LANES, LANES)]
            sy = syb[pl.ds(c * LANES, LANES)]
            x2 = x + sx
            y2 = y + sy
            bxl = x.astype(jnp.int32)
            byl = y.astype(jnp.int32)
            bxf = bxl.astype(jnp.float32)
            byf = byl.astype(jnp.float32)
            ox = [jnp.maximum(
                jnp.minimum(x2, bxf + (d + 1.0)) - jnp.maximum(x, bxf + float(d)),
                0.0) for d in range(3)]
            oy = [jnp.maximum(
                jnp.minimum(y2, byf + (d + 1.0)) - jnp.maximum(y, byf + float(d)),
                0.0) for d in range(3)]
            fb = bxl * NBX + byl
            r0 = c // 8
            col = (c % 8) * LANES
            for dx in range(3):
                for dy in range(3):
                    k = dx * 3 + dy
                    row = k * (B // 128) + r0
                    idxb[row, pl.ds(col, LANES)] = fb + (dx * NBX + dy)
                    wb[row, pl.ds(col, LANES)] = ox[dx] * oy[dy]
            return 0

        lax.fori_loop(0, B // LANES, gen, 0)

        pltpu.async_copy(map_sp.at[idxb], valb, sem).wait()

        def acc(c, _):
            r0 = c // 8
            col = (c % 8) * LANES
            s = jnp.zeros((LANES,), jnp.float32)
            for k in range(9):
                row = k * (B // 128) + r0
                s = s + valb[row, pl.ds(col, LANES)] * wb[row, pl.ds(col, LANES)]
            sx = sxb[pl.ds(c * LANES, LANES)]
            sy = syb[pl.ds(c * LANES, LANES)]
            pw = pwb[pl.ds(c * LANES, LANES)]
            outb[pl.ds(c * LANES, LANES)] = s * (10.0 * pw) / (sx * sy)
            return 0

        lax.fori_loop(0, B // LANES, acc, 0)

        pltpu.sync_copy(outb, out_hbm.at[pl.ds(base, B)])
        return 0

    lax.fori_loop(0, NBLK, block, 0)


@jax.jit
def _run(xs, ys, sxs, sys_, pws, map_flat):
    mesh = plsc.VectorSubcoreMesh(core_axis_name="c", subcore_axis_name="s")
    return pl.kernel(
        _body,
        out_type=jax.ShapeDtypeStruct((NP,), jnp.float32),
        mesh=mesh,
        scratch_types=[
            pltpu.VMEM_SHARED((MAP_WORDS,), jnp.float32),
            pltpu.VMEM((B,), jnp.float32),
            pltpu.VMEM((B,), jnp.float32),
            pltpu.VMEM((B,), jnp.float32),
            pltpu.VMEM((B,), jnp.float32),
            pltpu.VMEM((B,), jnp.float32),
            pltpu.VMEM((NR, 128), jnp.int32),
            pltpu.VMEM((NR, 128), jnp.float32),
            pltpu.VMEM((NR, 128), jnp.float32),
            pltpu.VMEM((B,), jnp.float32),
            pltpu.SemaphoreType.DMA,
        ],
    )(xs, ys, sxs, sys_, pws, map_flat)


def kernel(pos, node_size_x, node_size_y, utilization_map, pin_weights):
    n = N_NODES
    pad = NP - n
    x = jnp.concatenate([pos[:n], jnp.zeros((pad,), jnp.float32)])
    y = jnp.concatenate([pos[n:2 * n], jnp.zeros((pad,), jnp.float32)])
    sx = jnp.concatenate([node_size_x[:n], jnp.ones((pad,), jnp.float32)])
    sy = jnp.concatenate([node_size_y[:n], jnp.ones((pad,), jnp.float32)])
    pw = jnp.concatenate([pin_weights[:n], jnp.zeros((pad,), jnp.float32)])
    out = _run(x, y, sx, sy, pw, utilization_map.reshape(-1))
    return out[:n]


# speedup vs baseline: 3.0191x; 3.0191x over previous
import functools

import jax
import jax.numpy as jnp
from jax import lax
from jax.experimental import pallas as pl
from jax.experimental.pallas import tpu as pltpu
from jax.experimental.pallas import tpu_sc as plsc

N_NODES = 1000000
NBX = NBY = 1024
MAP_WORDS = NBX * NBY

NUM_CORES = 2
NUM_SUBCORES = 16
NW = NUM_CORES * NUM_SUBCORES
LANES = 16

B = 1664
NBLK = 19
C = B * NBLK
NP = NW * C
NR = 9 * B // 128


def _body(xs, ys, sxs, sys_, pws, map_hbm, out_hbm,
          map_sp, xb, yb, sxb, syb, pwb, idxb, wb, valb, outb, sem):
    cid = lax.axis_index("c")
    sid = lax.axis_index("s")
    wid = sid * NUM_CORES + cid

    @pl.when(sid == 0)
    def _():
        pltpu.sync_copy(map_hbm, map_sp)

    plsc.subcore_barrier()

    def block(blk, _):
        base = wid * C + blk * B
        pltpu.sync_copy(xs.at[pl.ds(base, B)], xb)
        pltpu.sync_copy(ys.at[pl.ds(base, B)], yb)
        pltpu.sync_copy(sxs.at[pl.ds(base, B)], sxb)
        pltpu.sync_copy(sys_.at[pl.ds(base, B)], syb)
        pltpu.sync_copy(pws.at[pl.ds(base, B)], pwb)

        def gen(c, _):
            x = xb[pl.ds(c * LANES, LANES)]
            y = yb[pl.ds(c * LANES, LANES)]
            sx = sxb[pl.ds(c * LANES, LANES)]
            sy = syb[pl.ds(c * LANES, LANES)]
            x2 = x + sx
            y2 = y + sy
            bxl = x.astype(jnp.int32)
            byl = y.astype(jnp.int32)
            bxf = bxl.astype(jnp.float32)
            byf = byl.astype(jnp.float32)
            ox = [jnp.maximum(
                jnp.minimum(x2, bxf + (d + 1.0)) - jnp.maximum(x, bxf + float(d)),
                0.0) for d in range(3)]
            oy = [jnp.maximum(
                jnp.minimum(y2, byf + (d + 1.0)) - jnp.maximum(y, byf + float(d)),
                0.0) for d in range(3)]
            fb = bxl * NBX + byl
            o = c * LANES
            for dx in range(3):
                for dy in range(3):
                    k = dx * 3 + dy
                    idxb[pl.ds(k * B + o, LANES)] = fb + (dx * NBX + dy)
                    wb[pl.ds(k * B + o, LANES)] = ox[dx] * oy[dy]
            return 0

        lax.fori_loop(0, B // LANES, gen, 0)

        pltpu.async_copy(map_sp.at[idxb], valb, sem).wait()

        def acc(c, _):
            o = c * LANES
            s = jnp.zeros((LANES,), jnp.float32)
            for k in range(9):
                s = s + valb[pl.ds(k * B + o, LANES)] * wb[pl.ds(k * B + o, LANES)]
            sx = sxb[pl.ds(c * LANES, LANES)]
            sy = syb[pl.ds(c * LANES, LANES)]
            pw = pwb[pl.ds(c * LANES, LANES)]
            outb[pl.ds(c * LANES, LANES)] = s * (10.0 * pw) / (sx * sy)
            return 0

        lax.fori_loop(0, B // LANES, acc, 0)

        pltpu.sync_copy(outb, out_hbm.at[pl.ds(base, B)])
        return 0

    lax.fori_loop(0, NBLK, block, 0)


@jax.jit
def _run(xs, ys, sxs, sys_, pws, map_flat):
    mesh = plsc.VectorSubcoreMesh(core_axis_name="c", subcore_axis_name="s")
    return pl.kernel(
        _body,
        out_type=jax.ShapeDtypeStruct((NP,), jnp.float32),
        mesh=mesh,
        scratch_types=[
            pltpu.VMEM_SHARED((MAP_WORDS,), jnp.float32),
            pltpu.VMEM((B,), jnp.float32),
            pltpu.VMEM((B,), jnp.float32),
            pltpu.VMEM((B,), jnp.float32),
            pltpu.VMEM((B,), jnp.float32),
            pltpu.VMEM((B,), jnp.float32),
            pltpu.VMEM((9 * B,), jnp.int32),
            pltpu.VMEM((9 * B,), jnp.float32),
            pltpu.VMEM((9 * B,), jnp.float32),
            pltpu.VMEM((B,), jnp.float32),
            pltpu.SemaphoreType.DMA,
        ],
    )(xs, ys, sxs, sys_, pws, map_flat)


def kernel(pos, node_size_x, node_size_y, utilization_map, pin_weights):
    n = N_NODES
    pad = NP - n
    x = jnp.concatenate([pos[:n], jnp.zeros((pad,), jnp.float32)])
    y = jnp.concatenate([pos[n:2 * n], jnp.zeros((pad,), jnp.float32)])
    sx = jnp.concatenate([node_size_x[:n], jnp.ones((pad,), jnp.float32)])
    sy = jnp.concatenate([node_size_y[:n], jnp.ones((pad,), jnp.float32)])
    pw = jnp.concatenate([pin_weights[:n], jnp.zeros((pad,), jnp.float32)])
    out = _run(x, y, sx, sy, pw, utilization_map.reshape(-1))
    return out[:n]
